# baseline (speedup 1.0000x reference)
import functools

import jax
import jax.numpy as jnp
from jax import lax
from jax.experimental import pallas as pl
from jax.experimental.pallas import tpu as pltpu
from jax.experimental.pallas import tpu_sc as plsc

N = 10000
E = 320000
D = 128
H = 128
C = 64
R = 8

NC = 2
NS = 16
NW = NC * NS
CE = 128
NCH0 = 128
NCH1 = 30
TOTCH = NS * (NCH0 + NCH1)
AGG_ROWS = 10240
ZROWS = AGG_ROWS // NS


def _sc_edge_agg_body(gidx_hbm, dst_hbm, hr_hbm, out_hbm,
                      gi_v, di_v, rows_v, agg_sh,
                      sem_g0, sem_g1, sem_i0, sem_i1, sem_d0, sem_d1):
    c = lax.axis_index("c")
    s = lax.axis_index("s")
    gsems = (sem_g0, sem_g1)
    isems = (sem_i0, sem_i1)
    dsems = (sem_d0, sem_d1)

    @pl.loop(0, CE)
    def _zrow(r):
        for j in range(H // 16):
            rows_v[0, r, pl.ds(j * 16, 16)] = jnp.zeros((16,), jnp.float32)

    @pl.loop(0, ZROWS // CE)
    def _zcopy(j):
        pltpu.sync_copy(rows_v.at[0], agg_sh.at[pl.ds(s * ZROWS + j * CE, CE)])

    plsc.subcore_barrier()

    def run(base, nch):
        def issue(ch, b):
            pltpu.async_copy(hr_hbm.at[gi_v.at[b]], rows_v.at[b], gsems[b])

        def wait_rows(ch, b):
            pltpu.make_async_copy(hr_hbm.at[gi_v.at[b]], rows_v.at[b],
                                  gsems[b]).wait()

        pltpu.sync_copy(gidx_hbm.at[base], gi_v.at[0])
        pltpu.sync_copy(gidx_hbm.at[base + 1], gi_v.at[1])
        pltpu.async_copy(dst_hbm.at[base], di_v.at[0], dsems[0])
        pltpu.async_copy(dst_hbm.at[base + 1], di_v.at[1], dsems[1])
        issue(0, 0)
        issue(1, 1)

        @pl.loop(0, nch - 2, step=2)
        def _main(ch0):
            for b in (0, 1):
                ch = ch0 + b
                wait_rows(ch, b)
                pltpu.async_copy(gidx_hbm.at[base + ch + 2], gi_v.at[b],
                                 isems[b])
                pltpu.make_async_copy(dst_hbm.at[base + ch], di_v.at[b],
                                      dsems[b]).wait()
                pltpu.sync_copy(rows_v.at[b], agg_sh.at[di_v.at[b]], add=True)
                pltpu.async_copy(dst_hbm.at[base + ch + 2], di_v.at[b],
                                 dsems[b])
                pltpu.make_async_copy(gidx_hbm.at[base + ch + 2], gi_v.at[b],
                                      isems[b]).wait()
                issue(ch + 2, b)

        for b in (0, 1):
            ch = nch - 2 + b
            wait_rows(ch, b)
            pltpu.make_async_copy(dst_hbm.at[base + ch], di_v.at[b],
                                  dsems[b]).wait()
            pltpu.sync_copy(rows_v.at[b], agg_sh.at[di_v.at[b]], add=True)

    @pl.when(c == 0)
    def _():
        run(s * NCH0, NCH0)

    @pl.when(c == 1)
    def _():
        run(NS * NCH0 + s * NCH1, NCH1)

    plsc.subcore_barrier()

    pltpu.sync_copy(agg_sh.at[pl.ds(s * ZROWS, ZROWS)],
                    out_hbm.at[c, pl.ds(s * ZROWS, ZROWS)])


_sc_edge_agg = pl.kernel(
    _sc_edge_agg_body,
    out_type=jax.ShapeDtypeStruct((NC, AGG_ROWS, H), jnp.float32),
    mesh=plsc.VectorSubcoreMesh(core_axis_name="c", subcore_axis_name="s",
                                num_cores=NC, num_subcores=NS),
    scratch_types=[
        pltpu.VMEM((2, CE), jnp.int32),
        pltpu.VMEM((2, CE), jnp.int32),
        pltpu.VMEM((2, CE, H), jnp.float32),
        pltpu.VMEM_SHARED((AGG_ROWS, H), jnp.float32),
        pltpu.SemaphoreType.DMA,
        pltpu.SemaphoreType.DMA,
        pltpu.SemaphoreType.DMA,
        pltpu.SemaphoreType.DMA,
        pltpu.SemaphoreType.DMA,
        pltpu.SemaphoreType.DMA,
    ],
)


BN = 1000
GRID = N // BN


def _mm1_body(x_ref, w_ref, ws_ref, b_ref, hr_ref, sp_ref):
    x = x_ref[...]
    for r in range(R):
        hr_ref[r] = jnp.dot(x, w_ref[r], preferred_element_type=jnp.float32)
    sp_ref[...] = jnp.dot(x, ws_ref[...],
                          preferred_element_type=jnp.float32) + b_ref[...]


_mm1 = pl.pallas_call(
    _mm1_body,
    grid=(GRID,),
    in_specs=[
        pl.BlockSpec((BN, D), lambda i: (i, 0)),
        pl.BlockSpec((R, D, H), lambda i: (0, 0, 0)),
        pl.BlockSpec((D, H), lambda i: (0, 0)),
        pl.BlockSpec((1, H), lambda i: (0, 0)),
    ],
    out_specs=[
        pl.BlockSpec((R, BN, H), lambda i: (0, i, 0)),
        pl.BlockSpec((BN, H), lambda i: (i, 0)),
    ],
    out_shape=[
        jax.ShapeDtypeStruct((R, N, H), jnp.float32),
        jax.ShapeDtypeStruct((N, H), jnp.float32),
    ],
)


def _mm2_body(p_ref, spin_ref, w_ref, ws_ref, b_ref, hr_ref, sp_ref):
    x = jnp.maximum(p_ref[0] + p_ref[1] + spin_ref[...], 0.0)
    for r in range(R):
        hr_ref[r] = jnp.dot(x, w_ref[r], preferred_element_type=jnp.float32)
    sp_ref[...] = jnp.dot(x, ws_ref[...],
                          preferred_element_type=jnp.float32) + b_ref[...]


_mm2 = pl.pallas_call(
    _mm2_body,
    grid=(GRID,),
    in_specs=[
        pl.BlockSpec((NC, BN, H), lambda i: (0, i, 0)),
        pl.BlockSpec((BN, H), lambda i: (i, 0)),
        pl.BlockSpec((R, H, H), lambda i: (0, 0, 0)),
        pl.BlockSpec((H, H), lambda i: (0, 0)),
        pl.BlockSpec((1, H), lambda i: (0, 0)),
    ],
    out_specs=[
        pl.BlockSpec((R, BN, H), lambda i: (0, i, 0)),
        pl.BlockSpec((BN, H), lambda i: (i, 0)),
    ],
    out_shape=[
        jax.ShapeDtypeStruct((R, N, H), jnp.float32),
        jax.ShapeDtypeStruct((N, H), jnp.float32),
    ],
)


def _fin_body(p_ref, spin_ref, wc_ref, bc_ref, o_ref, gm_ref):
    i = pl.program_id(0)
    y = jnp.maximum(p_ref[0] + p_ref[1] + spin_ref[...], 0.0)
    m = jnp.max(y, axis=0, keepdims=True)

    @pl.when(i == 0)
    def _():
        gm_ref[...] = m

    @pl.when(i > 0)
    def _():
        gm_ref[...] = jnp.maximum(gm_ref[...], m)

    @pl.when(i == pl.num_programs(0) - 1)
    def _():
        o_ref[...] = jnp.dot(gm_ref[...], wc_ref[...],
                             preferred_element_type=jnp.float32) + bc_ref[...]


_fin = pl.pallas_call(
    _fin_body,
    grid=(GRID,),
    in_specs=[
        pl.BlockSpec((NC, BN, H), lambda i: (0, i, 0)),
        pl.BlockSpec((BN, H), lambda i: (i, 0)),
        pl.BlockSpec((H, C), lambda i: (0, 0)),
        pl.BlockSpec((1, C), lambda i: (0, 0)),
    ],
    out_specs=pl.BlockSpec((1, C), lambda i: (0, 0)),
    out_shape=jax.ShapeDtypeStruct((1, C), jnp.float32),
    scratch_shapes=[pltpu.VMEM((1, H), jnp.float32)],
)


def kernel(h, edge_index, etype, W1, W1_self, b1, W2, W2_self, b2, Wc, bc):
    src = edge_index[0]
    dst = edge_index[1]
    pad = TOTCH * CE - E
    gidx = etype.astype(jnp.int32) * N + src
    gidx = gidx.reshape(E // R, R).T.reshape(-1)
    dstg = dst.reshape(E // R, R).T.reshape(-1)
    gidx_p = jnp.concatenate(
        [gidx, jnp.zeros((pad,), jnp.int32)]).reshape(TOTCH, CE)
    dst_p = jnp.concatenate(
        [dstg, jnp.full((pad,), N, jnp.int32)]).reshape(TOTCH, CE)

    hr1, sp1 = _mm1(h, W1, W1_self, b1.reshape(1, H))
    parts1 = _sc_edge_agg(gidx_p, dst_p, hr1.reshape(R * N, H))
    hr2, sp2 = _mm2(parts1, sp1, W2, W2_self, b2.reshape(1, H))
    parts2 = _sc_edge_agg(gidx_p, dst_p, hr2.reshape(R * N, H))
    return _fin(parts2, sp2, Wc, bc.reshape(1, C))


# speedup vs baseline: 1.2788x; 1.2788x over previous
import functools

import jax
import jax.numpy as jnp
from jax import lax
from jax.experimental import pallas as pl
from jax.experimental.pallas import tpu as pltpu
from jax.experimental.pallas import tpu_sc as plsc

N = 10000
E = 320000
D = 128
H = 128
C = 64
R = 8

NC = 2
NS = 16
NW = NC * NS
CE = 120
NCH0 = 138
NCH1 = 30
TOTCH = NS * (NCH0 + NCH1)
AGG_ROWS = 10112
ZROWS = AGG_ROWS // NS


def _sc_edge_agg_body(gidx_hbm, dst_hbm, hr_hbm, out_hbm,
                      gi_v, di_v, rows_v, agg_sh,
                      sem_g0, sem_g1, sem_g2, sem_i0, sem_i1, sem_i2,
                      sem_d0, sem_d1, sem_d2):
    c = lax.axis_index("c")
    s = lax.axis_index("s")
    gsems = (sem_g0, sem_g1, sem_g2)
    isems = (sem_i0, sem_i1, sem_i2)
    dsems = (sem_d0, sem_d1, sem_d2)

    @pl.loop(0, CE)
    def _zrow(r):
        for j in range(H // 16):
            rows_v[0, r, pl.ds(j * 16, 16)] = jnp.zeros((16,), jnp.float32)

    @pl.loop(0, ZROWS // CE)
    def _zcopy(j):
        pltpu.sync_copy(rows_v.at[0], agg_sh.at[pl.ds(s * ZROWS + j * CE, CE)])

    pltpu.sync_copy(rows_v.at[0].at[pl.ds(0, ZROWS % CE)],
                    agg_sh.at[pl.ds(s * ZROWS + (ZROWS // CE) * CE,
                                    ZROWS % CE)])

    plsc.subcore_barrier()

    def run(base, nch):
        def issue(ch, b):
            pltpu.async_copy(hr_hbm.at[gi_v.at[b]], rows_v.at[b], gsems[b])

        def wait_rows(ch, b):
            pltpu.make_async_copy(hr_hbm.at[gi_v.at[b]], rows_v.at[b],
                                  gsems[b]).wait()

        for b in (0, 1, 2):
            pltpu.sync_copy(gidx_hbm.at[base + b], gi_v.at[b])
            pltpu.async_copy(dst_hbm.at[base + b], di_v.at[b], dsems[b])
            issue(b, b)

        @pl.loop(0, nch - 3, step=3)
        def _main(ch0):
            for b in (0, 1, 2):
                ch = ch0 + b
                wait_rows(ch, b)
                pltpu.async_copy(gidx_hbm.at[base + ch + 3], gi_v.at[b],
                                 isems[b])
                pltpu.make_async_copy(dst_hbm.at[base + ch], di_v.at[b],
                                      dsems[b]).wait()
                pltpu.sync_copy(rows_v.at[b], agg_sh.at[di_v.at[b]], add=True)
                pltpu.async_copy(dst_hbm.at[base + ch + 3], di_v.at[b],
                                 dsems[b])
                pltpu.make_async_copy(gidx_hbm.at[base + ch + 3], gi_v.at[b],
                                      isems[b]).wait()
                issue(ch + 3, b)

        for b in (0, 1, 2):
            ch = nch - 3 + b
            wait_rows(ch, b)
            pltpu.make_async_copy(dst_hbm.at[base + ch], di_v.at[b],
                                  dsems[b]).wait()
            pltpu.sync_copy(rows_v.at[b], agg_sh.at[di_v.at[b]], add=True)

    @pl.when(c == 0)
    def _():
        run(s * NCH0, NCH0)

    @pl.when(c == 1)
    def _():
        run(NS * NCH0 + s * NCH1, NCH1)

    plsc.subcore_barrier()

    pltpu.sync_copy(agg_sh.at[pl.ds(s * ZROWS, ZROWS)],
                    out_hbm.at[c, pl.ds(s * ZROWS, ZROWS)])


_sc_edge_agg = pl.kernel(
    _sc_edge_agg_body,
    out_type=jax.ShapeDtypeStruct((NC, AGG_ROWS, H), jnp.float32),
    mesh=plsc.VectorSubcoreMesh(core_axis_name="c", subcore_axis_name="s",
                                num_cores=NC, num_subcores=NS),
    scratch_types=[
        pltpu.VMEM((3, CE), jnp.int32),
        pltpu.VMEM((3, CE), jnp.int32),
        pltpu.VMEM((3, CE, H), jnp.float32),
        pltpu.VMEM_SHARED((AGG_ROWS, H), jnp.float32),
        pltpu.SemaphoreType.DMA,
        pltpu.SemaphoreType.DMA,
        pltpu.SemaphoreType.DMA,
        pltpu.SemaphoreType.DMA,
        pltpu.SemaphoreType.DMA,
        pltpu.SemaphoreType.DMA,
        pltpu.SemaphoreType.DMA,
        pltpu.SemaphoreType.DMA,
        pltpu.SemaphoreType.DMA,
    ],
)


BN = 1000
GRID = N // BN


def _mm1_body(x_ref, w_ref, ws_ref, b_ref, hr_ref, sp_ref):
    x = x_ref[...]
    for r in range(R):
        hr_ref[r] = jnp.dot(x, w_ref[r], preferred_element_type=jnp.float32)
    sp_ref[...] = jnp.dot(x, ws_ref[...],
                          preferred_element_type=jnp.float32) + b_ref[...]


_mm1 = pl.pallas_call(
    _mm1_body,
    grid=(GRID,),
    in_specs=[
        pl.BlockSpec((BN, D), lambda i: (i, 0)),
        pl.BlockSpec((R, D, H), lambda i: (0, 0, 0)),
        pl.BlockSpec((D, H), lambda i: (0, 0)),
        pl.BlockSpec((1, H), lambda i: (0, 0)),
    ],
    out_specs=[
        pl.BlockSpec((R, BN, H), lambda i: (0, i, 0)),
        pl.BlockSpec((BN, H), lambda i: (i, 0)),
    ],
    out_shape=[
        jax.ShapeDtypeStruct((R, N, H), jnp.float32),
        jax.ShapeDtypeStruct((N, H), jnp.float32),
    ],
)


def _mm2_body(p_ref, spin_ref, w_ref, ws_ref, b_ref, hr_ref, sp_ref):
    x = jnp.maximum(p_ref[0] + p_ref[1] + spin_ref[...], 0.0)
    for r in range(R):
        hr_ref[r] = jnp.dot(x, w_ref[r], preferred_element_type=jnp.float32)
    sp_ref[...] = jnp.dot(x, ws_ref[...],
                          preferred_element_type=jnp.float32) + b_ref[...]


_mm2 = pl.pallas_call(
    _mm2_body,
    grid=(GRID,),
    in_specs=[
        pl.BlockSpec((NC, BN, H), lambda i: (0, i, 0)),
        pl.BlockSpec((BN, H), lambda i: (i, 0)),
        pl.BlockSpec((R, H, H), lambda i: (0, 0, 0)),
        pl.BlockSpec((H, H), lambda i: (0, 0)),
        pl.BlockSpec((1, H), lambda i: (0, 0)),
    ],
    out_specs=[
        pl.BlockSpec((R, BN, H), lambda i: (0, i, 0)),
        pl.BlockSpec((BN, H), lambda i: (i, 0)),
    ],
    out_shape=[
        jax.ShapeDtypeStruct((R, N, H), jnp.float32),
        jax.ShapeDtypeStruct((N, H), jnp.float32),
    ],
)


def _fin_body(p_ref, spin_ref, wc_ref, bc_ref, o_ref, gm_ref):
    i = pl.program_id(0)
    y = jnp.maximum(p_ref[0] + p_ref[1] + spin_ref[...], 0.0)
    m = jnp.max(y, axis=0, keepdims=True)

    @pl.when(i == 0)
    def _():
        gm_ref[...] = m

    @pl.when(i > 0)
    def _():
        gm_ref[...] = jnp.maximum(gm_ref[...], m)

    @pl.when(i == pl.num_programs(0) - 1)
    def _():
        o_ref[...] = jnp.dot(gm_ref[...], wc_ref[...],
                             preferred_element_type=jnp.float32) + bc_ref[...]


_fin = pl.pallas_call(
    _fin_body,
    grid=(GRID,),
    in_specs=[
        pl.BlockSpec((NC, BN, H), lambda i: (0, i, 0)),
        pl.BlockSpec((BN, H), lambda i: (i, 0)),
        pl.BlockSpec((H, C), lambda i: (0, 0)),
        pl.BlockSpec((1, C), lambda i: (0, 0)),
    ],
    out_specs=pl.BlockSpec((1, C), lambda i: (0, 0)),
    out_shape=jax.ShapeDtypeStruct((1, C), jnp.float32),
    scratch_shapes=[pltpu.VMEM((1, H), jnp.float32)],
)


def kernel(h, edge_index, etype, W1, W1_self, b1, W2, W2_self, b2, Wc, bc):
    src = edge_index[0]
    dst = edge_index[1]
    pad = TOTCH * CE - E
    gidx = etype.astype(jnp.int32) * N + src
    gidx_p = jnp.concatenate(
        [gidx, jnp.zeros((pad,), jnp.int32)]).reshape(TOTCH, CE)
    dst_p = jnp.concatenate(
        [dst, jnp.full((pad,), N, jnp.int32)]).reshape(TOTCH, CE)

    hr1, sp1 = _mm1(h, W1, W1_self, b1.reshape(1, H))
    parts1 = _sc_edge_agg(gidx_p, dst_p, hr1.reshape(R * N, H))
    hr2, sp2 = _mm2(parts1, sp1, W2, W2_self, b2.reshape(1, H))
    parts2 = _sc_edge_agg(gidx_p, dst_p, hr2.reshape(R * N, H))
    return _fin(parts2, sp2, Wc, bc.reshape(1, C))
